# baseline (speedup 1.0000x reference)
import functools

import jax
import jax.numpy as jnp
from jax import lax
from jax.experimental import pallas as pl
from jax.experimental.pallas import tpu as pltpu
from jax.experimental.pallas import tpu_sc as plsc

NUM_CORES = 2
NUM_SUBCORES = 16
NUM_WORKERS = NUM_CORES * NUM_SUBCORES
IDX_CHUNK = 128


@functools.partial(jax.jit, static_argnames=())
def _lookup(indices, embeds):
    (B,) = indices.shape
    V, D = embeds.shape
    b_per_w = B // NUM_WORKERS
    k = b_per_w // IDX_CHUNK

    mesh = plsc.VectorSubcoreMesh(core_axis_name="c", subcore_axis_name="s")

    @functools.partial(
        pl.kernel,
        mesh=mesh,
        out_type=jax.ShapeDtypeStruct((B, D), jnp.float32),
        compiler_params=pltpu.CompilerParams(use_tc_tiling_on_sc=False),
        scratch_types=[
            pltpu.VMEM((k, IDX_CHUNK), jnp.int32),
            pltpu.VMEM((b_per_w, D), jnp.float32),
            pltpu.SemaphoreType.DMA,
        ],
    )
    def body(idx_hbm, table_hbm, out_hbm, idx_v, rows_v, sem):
        wid = lax.axis_index("s") * NUM_CORES + lax.axis_index("c")
        base = wid * b_per_w
        for j in range(k):
            pltpu.sync_copy(
                idx_hbm.at[pl.ds(base + j * IDX_CHUNK, IDX_CHUNK)],
                idx_v.at[j],
            )
        copies = []
        for j in range(k):
            copies.append(
                pltpu.async_copy(
                    table_hbm.at[idx_v.at[j]],
                    rows_v.at[pl.ds(j * IDX_CHUNK, IDX_CHUNK)],
                    sem,
                )
            )
        for c in copies:
            c.wait()
        pltpu.sync_copy(rows_v, out_hbm.at[pl.ds(base, b_per_w)])

    return body(indices, embeds)


def kernel(indices, embeds):
    return _lookup(indices.astype(jnp.int32), embeds)


# speedup vs baseline: 1.6458x; 1.6458x over previous
import functools

import jax
import jax.numpy as jnp
from jax import lax
from jax.experimental import pallas as pl
from jax.experimental.pallas import tpu as pltpu
from jax.experimental.pallas import tpu_sc as plsc

NUM_CORES = 2
NUM_SUBCORES = 16
NUM_WORKERS = NUM_CORES * NUM_SUBCORES
K = 16


@jax.jit
def _lookup(indices, embeds):
    (B,) = indices.shape
    V, D = embeds.shape
    b_per_w = B // NUM_WORKERS

    mesh = plsc.VectorSubcoreMesh(core_axis_name="c", subcore_axis_name="s")

    @functools.partial(
        pl.kernel,
        mesh=mesh,
        out_type=jax.ShapeDtypeStruct((B, D), jnp.float32),
        scratch_types=[
            pltpu.VMEM((b_per_w,), jnp.int32),
            pltpu.VMEM((b_per_w, D), jnp.float32),
            pltpu.SemaphoreType.DMA,
        ],
    )
    def body(idx_hbm, tab_hbm, out_hbm, idx_v, rows_v, sem):
        wid = lax.axis_index("s") * NUM_CORES + lax.axis_index("c")
        base = wid * b_per_w
        pltpu.sync_copy(idx_hbm.at[pl.ds(base, b_per_w)], idx_v)

        def chunk(c, _):
            off = c * K
            vec = idx_v[pl.ds(off, K)]

            for j in range(K):
                pltpu.async_copy(
                    tab_hbm.at[pl.ds(vec[j], 1)],
                    rows_v.at[pl.ds(off + j, 1)],
                    sem,
                )

            def drain(j, _):
                pltpu.make_async_copy(
                    tab_hbm.at[pl.ds(0, 1)],
                    rows_v.at[pl.ds(off + j, 1)],
                    sem,
                ).wait()
                return 0

            lax.fori_loop(0, K, drain, 0)
            return 0

        lax.fori_loop(0, b_per_w // K, chunk, 0)
        pltpu.sync_copy(rows_v, out_hbm.at[pl.ds(base, b_per_w)])

    return body(indices, embeds)


def kernel(indices, embeds):
    return _lookup(indices.astype(jnp.int32), embeds)


# speedup vs baseline: 2.1340x; 1.2966x over previous
import functools

import jax
import jax.numpy as jnp
from jax import lax
from jax.experimental import pallas as pl
from jax.experimental.pallas import tpu as pltpu
from jax.experimental.pallas import tpu_sc as plsc

NUM_CORES = 2
NUM_SUBCORES = 16
NUM_WORKERS = NUM_CORES * NUM_SUBCORES
G = 4
LANES = 16


@jax.jit
def _lookup(indices, embeds):
    (B,) = indices.shape
    V, D = embeds.shape
    b_per_w = B // NUM_WORKERS
    tab_t = embeds.T

    mesh = plsc.VectorSubcoreMesh(core_axis_name="c", subcore_axis_name="s")

    @functools.partial(
        pl.kernel,
        mesh=mesh,
        out_type=jax.ShapeDtypeStruct((B, D), jnp.float32),
        scratch_types=[
            pltpu.VMEM((b_per_w,), jnp.int32),
            pltpu.VMEM((G, D, 128), jnp.float32),
            pltpu.VMEM((b_per_w, D), jnp.float32),
            pltpu.SemaphoreType.DMA,
        ],
        compiler_params=pltpu.CompilerParams(needs_layout_passes=False),
    )
    def body(idx_hbm, tab_hbm, out_hbm, idx_v, blocks_v, rows_v, sem):
        wid = lax.axis_index("s") * NUM_CORES + lax.axis_index("c")
        base = wid * b_per_w
        pltpu.sync_copy(idx_hbm.at[pl.ds(base, b_per_w)], idx_v)

        def chunk(c, _):
            off = c * LANES
            vec = idx_v[pl.ds(off, LANES)]
            qv = jnp.right_shift(vec, 7)
            ccv = jnp.bitwise_and(vec, 127)

            for part in range(LANES // G):
                for j in range(G):
                    jj = part * G + j
                    pltpu.async_copy(
                        tab_hbm.at[:, pl.ds(qv[jj] * 128, 128)],
                        blocks_v.at[j],
                        sem,
                    )
                for j in range(G):
                    jj = part * G + j
                    pltpu.make_async_copy(
                        tab_hbm.at[:, pl.ds(0, 128)],
                        blocks_v.at[j],
                        sem,
                    ).wait()
                    cc = jnp.full((LANES,), ccv[jj], dtype=jnp.int32)
                    for k in range(D // LANES):
                        dv = lax.iota(jnp.int32, LANES) + (k * LANES)
                        col = plsc.load_gather(blocks_v.at[j], [dv, cc])
                        rows_v[off + jj, pl.ds(k * LANES, LANES)] = col
            return 0

        lax.fori_loop(0, b_per_w // LANES, chunk, 0)
        pltpu.sync_copy(rows_v, out_hbm.at[pl.ds(base, b_per_w)])

    return body(indices, tab_t)


def kernel(indices, embeds):
    return _lookup(indices.astype(jnp.int32), embeds)


# speedup vs baseline: 2.6102x; 1.2232x over previous
import functools

import jax
import jax.numpy as jnp
from jax import lax
from jax.experimental import pallas as pl
from jax.experimental.pallas import tpu as pltpu
from jax.experimental.pallas import tpu_sc as plsc

NUM_CORES = 2
NUM_SUBCORES = 16
NUM_WORKERS = NUM_CORES * NUM_SUBCORES
G = 8
LANES = 16


@jax.jit
def _lookup(indices, embeds):
    (B,) = indices.shape
    V, D = embeds.shape
    b_per_w = B // NUM_WORKERS
    tab_t = embeds.T

    mesh = plsc.VectorSubcoreMesh(core_axis_name="c", subcore_axis_name="s")

    half_n = b_per_w // 2

    @functools.partial(
        pl.kernel,
        mesh=mesh,
        out_type=jax.ShapeDtypeStruct((B, D), jnp.float32),
        scratch_types=[
            pltpu.VMEM((b_per_w,), jnp.int32),
            pltpu.VMEM((G, D, 128), jnp.float32),
            pltpu.VMEM((half_n, D), jnp.float32),
            pltpu.SemaphoreType.DMA((G,)),
        ],
        compiler_params=pltpu.CompilerParams(needs_layout_passes=False),
    )
    def body(idx_hbm, tab_hbm, out_hbm, idx_v, blocks_v, rows_v, sems):
        wid = lax.axis_index("s") * NUM_CORES + lax.axis_index("c")
        base = wid * b_per_w
        pltpu.sync_copy(idx_hbm.at[pl.ds(base, b_per_w)], idx_v)

        for h in range(2):
            hb = h * half_n

            def chunk(c, _):
                off = c * LANES
                vec = idx_v[pl.ds(hb + off, LANES)]
                qv = jnp.right_shift(vec, 7)
                ccv = jnp.bitwise_and(vec, 127)

                def fire(i):
                    pltpu.async_copy(
                        tab_hbm.at[:, pl.ds(qv[i] * 128, 128)],
                        blocks_v.at[i % G],
                        sems.at[i % G],
                    )

                def proc(i):
                    pltpu.make_async_copy(
                        tab_hbm.at[:, pl.ds(0, 128)],
                        blocks_v.at[i % G],
                        sems.at[i % G],
                    ).wait()
                    cc = jnp.full((LANES,), ccv[i], dtype=jnp.int32)
                    for k in range(D // LANES):
                        dv = lax.iota(jnp.int32, LANES) + (k * LANES)
                        col = plsc.load_gather(blocks_v.at[i % G], [dv, cc])
                        rows_v[off + i, pl.ds(k * LANES, LANES)] = col

                for i in range(G):
                    fire(i)
                for i in range(G, LANES):
                    proc(i - G)
                    fire(i)
                for i in range(LANES - G, LANES):
                    proc(i)
                return 0

            lax.fori_loop(0, half_n // LANES, chunk, 0)
            pltpu.sync_copy(rows_v, out_hbm.at[pl.ds(base + hb, half_n)])

    return body(indices, tab_t)


def kernel(indices, embeds):
    return _lookup(indices.astype(jnp.int32), embeds)


# speedup vs baseline: 2.7612x; 1.0578x over previous
import functools

import jax
import jax.numpy as jnp
from jax import lax
from jax.experimental import pallas as pl
from jax.experimental.pallas import tpu as pltpu
from jax.experimental.pallas import tpu_sc as plsc

NUM_CORES = 2
NUM_SUBCORES = 16
NUM_WORKERS = NUM_CORES * NUM_SUBCORES
G = 11
LANES = 16
CHUNK = 32


@jax.jit
def _lookup(indices, embeds):
    (B,) = indices.shape
    V, D = embeds.shape
    b_per_w = B // NUM_WORKERS
    tab_t = embeds.T

    mesh = plsc.VectorSubcoreMesh(core_axis_name="c", subcore_axis_name="s")

    half_n = b_per_w // 2

    @functools.partial(
        pl.kernel,
        mesh=mesh,
        out_type=jax.ShapeDtypeStruct((B, D), jnp.float32),
        scratch_types=[
            pltpu.VMEM((b_per_w,), jnp.int32),
            pltpu.VMEM((G, D, 128), jnp.float32),
            pltpu.VMEM((half_n, D), jnp.float32),
            pltpu.SemaphoreType.DMA((G,)),
        ],
        compiler_params=pltpu.CompilerParams(needs_layout_passes=False),
    )
    def body(idx_hbm, tab_hbm, out_hbm, idx_v, blocks_v, rows_v, sems):
        wid = lax.axis_index("s") * NUM_CORES + lax.axis_index("c")
        base = wid * b_per_w
        pltpu.sync_copy(idx_hbm.at[pl.ds(base, b_per_w)], idx_v)

        for h in range(2):
            hb = h * half_n

            def chunk(c, _):
                off = c * CHUNK
                vecs = [
                    idx_v[pl.ds(hb + off + v * LANES, LANES)]
                    for v in range(CHUNK // LANES)
                ]
                qvs = [jnp.right_shift(v, 7) for v in vecs]
                ccvs = [jnp.bitwise_and(v, 127) for v in vecs]

                def fire(i):
                    pltpu.async_copy(
                        tab_hbm.at[:, pl.ds(qvs[i // LANES][i % LANES] * 128, 128)],
                        blocks_v.at[i % G],
                        sems.at[i % G],
                    )

                def proc(i):
                    pltpu.make_async_copy(
                        tab_hbm.at[:, pl.ds(0, 128)],
                        blocks_v.at[i % G],
                        sems.at[i % G],
                    ).wait()
                    cc = jnp.full(
                        (LANES,), ccvs[i // LANES][i % LANES], dtype=jnp.int32
                    )
                    for k in range(D // LANES):
                        dv = lax.iota(jnp.int32, LANES) + (k * LANES)
                        col = plsc.load_gather(blocks_v.at[i % G], [dv, cc])
                        rows_v[off + i, pl.ds(k * LANES, LANES)] = col

                for i in range(G):
                    fire(i)
                for i in range(G, CHUNK):
                    proc(i - G)
                    fire(i)
                for i in range(CHUNK - G, CHUNK):
                    proc(i)
                return 0

            lax.fori_loop(0, half_n // CHUNK, chunk, 0)
            pltpu.sync_copy(rows_v, out_hbm.at[pl.ds(base + hb, half_n)])

    return body(indices, tab_t)


def kernel(indices, embeds):
    return _lookup(indices.astype(jnp.int32), embeds)


# speedup vs baseline: 2.8416x; 1.0291x over previous
import functools

import jax
import jax.numpy as jnp
from jax import lax
from jax.experimental import pallas as pl
from jax.experimental.pallas import tpu as pltpu
from jax.experimental.pallas import tpu_sc as plsc

NUM_CORES = 2
NUM_SUBCORES = 16
NUM_WORKERS = NUM_CORES * NUM_SUBCORES
G = 11
LANES = 16
CHUNK = 64


@jax.jit
def _lookup(indices, embeds):
    (B,) = indices.shape
    V, D = embeds.shape
    b_per_w = B // NUM_WORKERS
    tab_t = embeds.T

    mesh = plsc.VectorSubcoreMesh(core_axis_name="c", subcore_axis_name="s")

    half_n = b_per_w // 2

    @functools.partial(
        pl.kernel,
        mesh=mesh,
        out_type=jax.ShapeDtypeStruct((B, D), jnp.float32),
        scratch_types=[
            pltpu.VMEM((b_per_w,), jnp.int32),
            pltpu.VMEM((G, D, 128), jnp.float32),
            pltpu.VMEM((half_n, D), jnp.float32),
            pltpu.SemaphoreType.DMA((G,)),
        ],
        compiler_params=pltpu.CompilerParams(needs_layout_passes=False),
    )
    def body(idx_hbm, tab_hbm, out_hbm, idx_v, blocks_v, rows_v, sems):
        wid = lax.axis_index("s") * NUM_CORES + lax.axis_index("c")
        base = wid * b_per_w
        pltpu.sync_copy(idx_hbm.at[pl.ds(base, b_per_w)], idx_v)

        for h in range(2):
            hb = h * half_n

            def chunk(c, _):
                off = c * CHUNK
                vecs = [
                    idx_v[pl.ds(hb + off + v * LANES, LANES)]
                    for v in range(CHUNK // LANES)
                ]
                qvs = [jnp.right_shift(v, 7) for v in vecs]
                ccvs = [jnp.bitwise_and(v, 127) for v in vecs]

                def fire(i):
                    pltpu.async_copy(
                        tab_hbm.at[:, pl.ds(qvs[i // LANES][i % LANES] * 128, 128)],
                        blocks_v.at[i % G],
                        sems.at[i % G],
                    )

                def proc(i):
                    pltpu.make_async_copy(
                        tab_hbm.at[:, pl.ds(0, 128)],
                        blocks_v.at[i % G],
                        sems.at[i % G],
                    ).wait()
                    cc = jnp.full(
                        (LANES,), ccvs[i // LANES][i % LANES], dtype=jnp.int32
                    )
                    for k in range(D // LANES):
                        dv = lax.iota(jnp.int32, LANES) + (k * LANES)
                        col = plsc.load_gather(blocks_v.at[i % G], [dv, cc])
                        rows_v[off + i, pl.ds(k * LANES, LANES)] = col

                for i in range(G):
                    fire(i)
                for i in range(G, CHUNK):
                    proc(i - G)
                    fire(i)
                for i in range(CHUNK - G, CHUNK):
                    proc(i)
                return 0

            lax.fori_loop(0, half_n // CHUNK, chunk, 0)
            pltpu.sync_copy(rows_v, out_hbm.at[pl.ds(base + hb, half_n)])

    return body(indices, tab_t)


def kernel(indices, embeds):
    return _lookup(indices.astype(jnp.int32), embeds)
